# Initial kernel scaffold; baseline (speedup 1.0000x reference)
import functools

import jax
import jax.numpy as jnp
from jax import lax
from jax.experimental import pallas as pl
from jax.experimental.pallas import tpu as pltpu
from jax.experimental.pallas import tpu_sc as plsc

VOCAB = 100000
EMBED_DIM = 64
NUM_WORKERS = 32
CHUNK = 128
TOTAL = 4096 * 26
PER_WORKER = TOTAL // NUM_WORKERS
CHUNKS_PER_WORKER = PER_WORKER // CHUNK

_MESH = plsc.VectorSubcoreMesh(core_axis_name="c", subcore_axis_name="s")


@functools.partial(
    pl.kernel,
    out_type=jax.ShapeDtypeStruct((TOTAL, EMBED_DIM), jnp.float32),
    mesh=_MESH,
    scratch_types=[
        pltpu.VMEM((CHUNKS_PER_WORKER, CHUNK), jnp.int32),
        pltpu.VMEM((CHUNK, EMBED_DIM), jnp.float32),
        pltpu.VMEM((CHUNK, EMBED_DIM), jnp.float32),
        pltpu.SemaphoreType.DMA,
        pltpu.SemaphoreType.DMA,
    ],
)
def _gather_kernel(table_hbm, idx_hbm, out_hbm, idx_v, buf_a, buf_b, sem_a, sem_b):
    wid = lax.axis_index("s") * 2 + lax.axis_index("c")
    base = wid * PER_WORKER

    pltpu.sync_copy(idx_hbm.at[pl.ds(wid * CHUNKS_PER_WORKER, CHUNKS_PER_WORKER)],
                    idx_v)

    bufs = (buf_a, buf_b)
    sems = (sem_a, sem_b)

    copies = [None, None]
    copies[0] = pltpu.async_copy(table_hbm.at[idx_v.at[0]], bufs[0], sems[0])
    for j in range(CHUNKS_PER_WORKER):
        nxt = j + 1
        if nxt < CHUNKS_PER_WORKER:
            copies[nxt % 2] = pltpu.async_copy(
                table_hbm.at[idx_v.at[nxt]], bufs[nxt % 2], sems[nxt % 2])
        copies[j % 2].wait()
        pltpu.sync_copy(bufs[j % 2],
                        out_hbm.at[pl.ds(base + j * CHUNK, CHUNK)])


def kernel(indices, table):
    idx = indices.astype(jnp.int32).reshape(TOTAL // CHUNK, CHUNK)
    out = _gather_kernel(table, idx)
    return out.reshape(4096, 26, EMBED_DIM)


# speedup vs baseline: 1.1839x; 1.1839x over previous
import functools

import jax
import jax.numpy as jnp
from jax import lax
from jax.experimental import pallas as pl
from jax.experimental.pallas import tpu as pltpu
from jax.experimental.pallas import tpu_sc as plsc

VOCAB = 100000
EMBED_DIM = 64
NUM_WORKERS = 32
CHUNK = 128
TOTAL = 4096 * 26
PER_WORKER = TOTAL // NUM_WORKERS
CHUNKS_PER_WORKER = PER_WORKER // CHUNK

_MESH = plsc.VectorSubcoreMesh(core_axis_name="c", subcore_axis_name="s")


@functools.partial(
    pl.kernel,
    out_type=jax.ShapeDtypeStruct((TOTAL, EMBED_DIM), jnp.float32),
    mesh=_MESH,
    compiler_params=pltpu.CompilerParams(use_tc_tiling_on_sc=False),
    scratch_types=[
        pltpu.VMEM((CHUNKS_PER_WORKER, CHUNK), jnp.int32),
        pltpu.VMEM((CHUNK, EMBED_DIM), jnp.float32),
        pltpu.VMEM((CHUNK, EMBED_DIM), jnp.float32),
        pltpu.SemaphoreType.DMA,
        pltpu.SemaphoreType.DMA,
    ],
)
def _gather_kernel(table_hbm, idx_hbm, out_hbm, idx_v, buf_a, buf_b, sem_a, sem_b):
    wid = lax.axis_index("s") * 2 + lax.axis_index("c")
    base = wid * PER_WORKER

    pltpu.sync_copy(idx_hbm.at[wid], idx_v)

    bufs = (buf_a, buf_b)
    sems = (sem_a, sem_b)

    copies = [None, None]
    copies[0] = pltpu.async_copy(table_hbm.at[idx_v.at[0]], bufs[0], sems[0])
    for j in range(CHUNKS_PER_WORKER):
        nxt = j + 1
        if nxt < CHUNKS_PER_WORKER:
            copies[nxt % 2] = pltpu.async_copy(
                table_hbm.at[idx_v.at[nxt]], bufs[nxt % 2], sems[nxt % 2])
        copies[j % 2].wait()
        pltpu.sync_copy(bufs[j % 2],
                        out_hbm.at[pl.ds(base + j * CHUNK, CHUNK)])


def kernel(indices, table):
    idx = indices.astype(jnp.int32).reshape(NUM_WORKERS, CHUNKS_PER_WORKER, CHUNK)
    out = _gather_kernel(table, idx)
    return out.reshape(4096, 26, EMBED_DIM)


# speedup vs baseline: 1.2014x; 1.0148x over previous
import functools

import jax
import jax.numpy as jnp
from jax import lax
from jax.experimental import pallas as pl
from jax.experimental.pallas import tpu as pltpu
from jax.experimental.pallas import tpu_sc as plsc

VOCAB = 100000
EMBED_DIM = 64
NUM_WORKERS = 32
CHUNK = 128
TOTAL = 4096 * 26
PER_WORKER = TOTAL // NUM_WORKERS
CHUNKS_PER_WORKER = PER_WORKER // CHUNK

_MESH = plsc.VectorSubcoreMesh(core_axis_name="c", subcore_axis_name="s")


@functools.partial(
    pl.kernel,
    out_type=jax.ShapeDtypeStruct((TOTAL, EMBED_DIM), jnp.float32),
    mesh=_MESH,
    compiler_params=pltpu.CompilerParams(use_tc_tiling_on_sc=False),
    scratch_types=[
        pltpu.VMEM((CHUNKS_PER_WORKER, CHUNK), jnp.int32),
        pltpu.VMEM((4, CHUNK, EMBED_DIM), jnp.float32),
        pltpu.SemaphoreType.DMA,
        pltpu.SemaphoreType.DMA,
        pltpu.SemaphoreType.DMA,
        pltpu.SemaphoreType.DMA,
        pltpu.SemaphoreType.DMA,
        pltpu.SemaphoreType.DMA,
        pltpu.SemaphoreType.DMA,
        pltpu.SemaphoreType.DMA,
    ],
)
def _gather_kernel(table_hbm, idx_hbm, out_hbm, idx_v, bufs,
                   g0, g1, g2, g3, w0, w1, w2, w3):
    wid = lax.axis_index("s") * 2 + lax.axis_index("c")
    base = wid * PER_WORKER
    NBUF = 4
    gsems = (g0, g1, g2, g3)
    wsems = (w0, w1, w2, w3)

    pltpu.sync_copy(idx_hbm.at[wid], idx_v)

    gc = [None] * NBUF
    wc = [None] * NBUF
    for j in range(CHUNKS_PER_WORKER + 1):
        if j < CHUNKS_PER_WORKER:
            b = j % NBUF
            if j >= NBUF:
                wc[b].wait()
            gc[b] = pltpu.async_copy(table_hbm.at[idx_v.at[j]],
                                     bufs.at[b], gsems[b])
        if j >= 1:
            p = j - 1
            b = p % NBUF
            gc[b].wait()
            wc[b] = pltpu.async_copy(
                bufs.at[b], out_hbm.at[pl.ds(base + p * CHUNK, CHUNK)],
                wsems[b])
    for b in range(NBUF):
        wc[b].wait()


def kernel(indices, table):
    idx = indices.astype(jnp.int32).reshape(NUM_WORKERS, CHUNKS_PER_WORKER, CHUNK)
    out = _gather_kernel(table, idx)
    return out.reshape(4096, 26, EMBED_DIM)


# speedup vs baseline: 1.2065x; 1.0042x over previous
import functools

import jax
import jax.numpy as jnp
from jax import lax
from jax.experimental import pallas as pl
from jax.experimental.pallas import tpu as pltpu
from jax.experimental.pallas import tpu_sc as plsc

VOCAB = 100000
EMBED_DIM = 64
BATCH = 4096
SEQ = 26
NUM_WORKERS = 32
SLABS_PER_WORKER = BATCH // NUM_WORKERS
GROUP = 16
GROUPS_PER_WORKER = SLABS_PER_WORKER // GROUP

_MESH = plsc.VectorSubcoreMesh(core_axis_name="c", subcore_axis_name="s")


@functools.partial(
    pl.kernel,
    out_type=jax.ShapeDtypeStruct((BATCH, SEQ, EMBED_DIM), jnp.float32),
    mesh=_MESH,
    compiler_params=pltpu.CompilerParams(use_tc_tiling_on_sc=False),
    scratch_types=[
        pltpu.VMEM((SLABS_PER_WORKER, SEQ), jnp.int32),
        pltpu.VMEM((GROUP, SEQ, EMBED_DIM), jnp.float32),
        pltpu.VMEM((GROUP, SEQ, EMBED_DIM), jnp.float32),
        pltpu.SemaphoreType.DMA,
        pltpu.SemaphoreType.DMA,
        pltpu.SemaphoreType.DMA,
        pltpu.SemaphoreType.DMA,
    ],
)
def _gather_kernel(table_hbm, idx_hbm, out_hbm, idx_v, buf0, buf1,
                   g0, g1, w0, w1):
    wid = lax.axis_index("s") * 2 + lax.axis_index("c")
    sbase = wid * SLABS_PER_WORKER

    pltpu.sync_copy(idx_hbm.at[pl.ds(sbase, SLABS_PER_WORKER)], idx_v)

    bufs = (buf0, buf1)
    gsems = (g0, g1)
    wsems = (w0, w1)

    def body(i, _):
        gcopies = [[None] * GROUP, [None] * GROUP]
        for p in range(2):
            g = 2 * i + p

            @pl.when(i > 0)
            def _():
                pltpu.make_async_copy(out_hbm.at[pl.ds(0, GROUP)], bufs[p],
                                      wsems[p]).wait()

            for s in range(GROUP):
                gcopies[p][s] = pltpu.async_copy(
                    table_hbm.at[idx_v.at[g * GROUP + s]],
                    bufs[p].at[s], gsems[p])
        for p in range(2):
            g = 2 * i + p
            for s in range(GROUP):
                gcopies[p][s].wait()
            pltpu.async_copy(bufs[p],
                             out_hbm.at[pl.ds(sbase + g * GROUP, GROUP)],
                             wsems[p])
        return _

    lax.fori_loop(0, GROUPS_PER_WORKER // 2, body, None)

    for p in range(2):
        pltpu.make_async_copy(out_hbm.at[pl.ds(0, GROUP)], bufs[p],
                              wsems[p]).wait()


def kernel(indices, table):
    idx = indices.astype(jnp.int32)
    return _gather_kernel(table, idx)
